# Initial kernel scaffold; baseline (speedup 1.0000x reference)
import jax
import jax.numpy as jnp
from jax.experimental import pallas as pl


def kernel(x, edge_index, W, b):
    raise NotImplementedError("write your pallas kernel here")


# speedup vs baseline: 11.7415x; 11.7415x over previous
import functools

import jax
import jax.numpy as jnp
from jax import lax
from jax.experimental import pallas as pl
from jax.experimental.pallas import tpu as pltpu
from jax.experimental.pallas import tpu_sc as plsc

NC = 2
NS = 16
LANES = 16

CH = 128
UNROLL = 4


def _ceil_div(a, b):
    return (a + b - 1) // b


def _mesh():
    return plsc.VectorSubcoreMesh(core_axis_name="c", subcore_axis_name="s",
                                  num_cores=NC, num_subcores=NS)


def _zero_rows(buf, ref, row0, rows):
    full, rem = divmod(rows, CH)

    def chunk(j, _):
        pltpu.sync_copy(buf, ref.at[pl.ds(row0 + j * CH, CH)])
        return 0

    lax.fori_loop(0, full, chunk, 0)
    if rem:
        pltpu.sync_copy(buf.at[pl.ds(0, rem)],
                        ref.at[pl.ds(row0 + full * CH, rem)])


def _tile_range(total, workers, wid):
    lo = total // workers
    extra = total % workers
    start = wid * lo + jnp.minimum(wid, extra)
    count = lo + jnp.where(wid < extra, 1, 0)
    return start, count, lo + (1 if extra else 0)


def _make_deg_kernel(nchunks, half, deg_rows):
    lo = nchunks // NS
    max_ct = lo + (1 if nchunks % NS else 0)
    main = (lo // UNROLL) * UNROLL
    tile_rows = deg_rows // NS
    trash = half

    @functools.partial(
        pl.kernel,
        out_type=jax.ShapeDtypeStruct((NC * deg_rows, LANES), jnp.float32),
        mesh=_mesh(),
        compiler_params=pltpu.CompilerParams(use_tc_tiling_on_sc=False),
        scratch_types=[
            pltpu.VMEM((UNROLL, CH), jnp.int32),
            pltpu.VMEM((CH, LANES), jnp.float32),
            pltpu.VMEM((CH, LANES), jnp.float32),
            pltpu.VMEM_SHARED((deg_rows, LANES), jnp.float32),
        ] + [pltpu.SemaphoreType.DMA] * (2 * UNROLL),
    )
    def deg_kernel(dst_hbm, deg_out, idx_buf, ones_buf, zeros_buf, deg_sp,
                   *sems):
        isems = sems[:UNROLL]
        ssems = sems[UNROLL:]
        cid = lax.axis_index("c")
        sid = lax.axis_index("s")
        base = cid * half
        start, count, _ = _tile_range(nchunks, NS, sid)

        ones = jnp.full((LANES,), 1.0, dtype=jnp.float32)
        zeros = jnp.zeros((LANES,), dtype=jnp.float32)

        def init_row(r, _):
            ones_buf[r, :] = ones
            zeros_buf[r, :] = zeros
            return 0

        lax.fori_loop(0, CH, init_row, 0)
        _zero_rows(zeros_buf, deg_sp, sid * tile_rows, tile_rows)
        plsc.subcore_barrier()

        def remap_row(u):
            for q in range(CH // LANES):
                dloc = idx_buf[u, pl.ds(q * LANES, LANES)] - base
                ok = (dloc >= 0) & (dloc < half)
                idx_buf[u, pl.ds(q * LANES, LANES)] = jnp.where(ok, dloc,
                                                                trash)

        def step(t, _):
            c0 = (start + t * UNROLL) * CH
            ldescs = [
                pltpu.async_copy(dst_hbm.at[pl.ds(c0 + u * CH, CH)],
                                 idx_buf.at[u], isems[u])
                for u in range(UNROLL)
            ]
            sdescs = []
            for u in range(UNROLL):
                ldescs[u].wait()
                remap_row(u)
                sdescs.append(pltpu.async_copy(
                    ones_buf, deg_sp.at[idx_buf.at[u]], ssems[u], add=True))
            for sd in sdescs:
                sd.wait()
            return 0

        lax.fori_loop(0, main // UNROLL, step, 0)
        for j in range(main, max_ct):
            @pl.when(j < count)
            def _():
                pltpu.sync_copy(dst_hbm.at[pl.ds((start + j) * CH, CH)],
                                idx_buf.at[0])
                remap_row(0)
                pltpu.sync_copy(ones_buf, deg_sp.at[idx_buf.at[0]], add=True)
        plsc.subcore_barrier()

        pltpu.sync_copy(
            deg_sp.at[pl.ds(sid * tile_rows, tile_rows)],
            deg_out.at[pl.ds(cid * deg_rows + sid * tile_rows, tile_rows)])

    return deg_kernel


def _make_edge_kernel(nchunks, half, acc_rows, d):
    lo = nchunks // NS
    max_ct = lo + (1 if nchunks % NS else 0)
    main = (lo // UNROLL) * UNROLL
    tile_rows = acc_rows // NS
    trash = half

    @functools.partial(
        pl.kernel,
        out_type=jax.ShapeDtypeStruct((NC * acc_rows, d), jnp.bfloat16),
        mesh=_mesh(),
        compiler_params=pltpu.CompilerParams(use_tc_tiling_on_sc=False),
        scratch_types=[
            pltpu.VMEM((UNROLL, CH), jnp.int32),
            pltpu.VMEM((UNROLL, CH), jnp.int32),
            pltpu.VMEM((CH, d), jnp.bfloat16),
            pltpu.VMEM_SHARED((acc_rows, d), jnp.bfloat16),
        ] + [pltpu.VMEM((CH, d), jnp.bfloat16)] * UNROLL
          + [pltpu.SemaphoreType.DMA] * (3 * UNROLL),
    )
    def edge_kernel(src_hbm, dst_hbm, gbf_hbm, acc_out,
                    src_buf, dst_buf, zeros_buf, acc_sp, *rows_and_sems):
        rows = rows_and_sems[:UNROLL]
        isems = rows_and_sems[UNROLL:2 * UNROLL]
        gsems = rows_and_sems[2 * UNROLL:3 * UNROLL]
        ssems = rows_and_sems[3 * UNROLL:]
        cid = lax.axis_index("c")
        sid = lax.axis_index("s")
        base = cid * half
        start, count, _ = _tile_range(nchunks, NS, sid)

        zeros = jnp.zeros((2 * LANES,), dtype=jnp.bfloat16)

        def init_row(r, _):
            for q in range(d // (2 * LANES)):
                zeros_buf[r, pl.ds(q * 2 * LANES, 2 * LANES)] = zeros
            return 0

        lax.fori_loop(0, CH, init_row, 0)
        _zero_rows(zeros_buf, acc_sp, sid * tile_rows, tile_rows)
        plsc.subcore_barrier()

        def remap_row(u):
            for q in range(CH // LANES):
                dloc = dst_buf[u, pl.ds(q * LANES, LANES)] - base
                ok = (dloc >= 0) & (dloc < half)
                dst_buf[u, pl.ds(q * LANES, LANES)] = jnp.where(ok, dloc,
                                                                trash)

        def step(t, _):
            c0 = (start + t * UNROLL) * CH
            ldescs = []
            for u in range(UNROLL):
                ldescs.append(pltpu.async_copy(
                    src_hbm.at[pl.ds(c0 + u * CH, CH)],
                    src_buf.at[u], isems[u]))
                pltpu.sync_copy(dst_hbm.at[pl.ds(c0 + u * CH, CH)],
                                dst_buf.at[u])
            gdescs = []
            for u in range(UNROLL):
                ldescs[u].wait()
                gdescs.append(pltpu.async_copy(
                    gbf_hbm.at[src_buf.at[u]], rows[u], gsems[u]))
            sdescs = []
            for u in range(UNROLL):
                remap_row(u)
                gdescs[u].wait()
                sdescs.append(pltpu.async_copy(
                    rows[u], acc_sp.at[dst_buf.at[u]], ssems[u], add=True))
            for sd in sdescs:
                sd.wait()
            return 0

        lax.fori_loop(0, main // UNROLL, step, 0)
        for j in range(main, max_ct):
            @pl.when(j < count)
            def _():
                pltpu.sync_copy(src_hbm.at[pl.ds((start + j) * CH, CH)],
                                src_buf.at[0])
                pltpu.sync_copy(dst_hbm.at[pl.ds((start + j) * CH, CH)],
                                dst_buf.at[0])
                remap_row(0)
                pltpu.async_copy(gbf_hbm.at[src_buf.at[0]], rows[0],
                                 gsems[0]).wait()
                pltpu.sync_copy(rows[0], acc_sp.at[dst_buf.at[0]], add=True)
        plsc.subcore_barrier()

        pltpu.sync_copy(
            acc_sp.at[pl.ds(sid * tile_rows, tile_rows)],
            acc_out.at[pl.ds(cid * acc_rows + sid * tile_rows, tile_rows)])

    return edge_kernel


def _pre_body(x_ref, w_ref, deg_ref, g_ref, gbf_ref, dinv_ref):
    deg = deg_ref[...] + 1.0
    dinv = lax.rsqrt(deg)
    dinv_ref[...] = dinv
    g = jnp.dot(x_ref[...], w_ref[...],
                preferred_element_type=jnp.float32) * dinv
    g_ref[...] = g
    gbf_ref[...] = g.astype(jnp.bfloat16)


def _post_body(acc_ref, g_ref, dinv_ref, b_ref, o_ref):
    acc = acc_ref[...].astype(jnp.float32)
    o_ref[...] = jnp.tanh((acc + g_ref[...]) * dinv_ref[...] + b_ref[...])


BN = 128


def kernel(x, edge_index, W, b):
    n, d_in = x.shape
    d_out = W.shape[1]
    e = edge_index.shape[1]

    nb = _ceil_div(n, BN)
    npad = nb * BN

    assert e % CH == 0 and (e // CH) % NC == 0 and n % 2 == 0
    nchunks = e // CH
    half = n // 2
    deg_rows = _ceil_div(half + 1, NS) * NS
    acc_rows = deg_rows

    src = edge_index[0]
    dst = edge_index[1]

    deg_kernel = _make_deg_kernel(nchunks, half, deg_rows)
    deg_parts = deg_kernel(dst)
    degw = deg_parts.reshape(NC, deg_rows, LANES)
    deg = jnp.concatenate([degw[0, :half, 0:1], degw[1, :half, 0:1]], axis=0)

    g, gbf, dinv = pl.pallas_call(
        _pre_body,
        grid=(nb,),
        in_specs=[
            pl.BlockSpec((BN, d_in), lambda i: (i, 0)),
            pl.BlockSpec((d_in, d_out), lambda i: (0, 0)),
            pl.BlockSpec((BN, 1), lambda i: (i, 0)),
        ],
        out_specs=[
            pl.BlockSpec((BN, d_out), lambda i: (i, 0)),
            pl.BlockSpec((BN, d_out), lambda i: (i, 0)),
            pl.BlockSpec((BN, 1), lambda i: (i, 0)),
        ],
        out_shape=[
            jax.ShapeDtypeStruct((npad, d_out), jnp.float32),
            jax.ShapeDtypeStruct((npad, d_out), jnp.bfloat16),
            jax.ShapeDtypeStruct((npad, 1), jnp.float32),
        ],
    )(x, W, deg)

    edge_kernel = _make_edge_kernel(nchunks, half, acc_rows, d_out)
    accp = edge_kernel(src, dst, gbf).reshape(NC, acc_rows, d_out)
    acc = jnp.concatenate([accp[0, :half], accp[1, :half]], axis=0)

    out = pl.pallas_call(
        _post_body,
        grid=(nb,),
        in_specs=[
            pl.BlockSpec((BN, d_out), lambda i: (i, 0)),
            pl.BlockSpec((BN, d_out), lambda i: (i, 0)),
            pl.BlockSpec((BN, 1), lambda i: (i, 0)),
            pl.BlockSpec((1, d_out), lambda i: (0, 0)),
        ],
        out_specs=pl.BlockSpec((BN, d_out), lambda i: (i, 0)),
        out_shape=jax.ShapeDtypeStruct((n, d_out), jnp.float32),
    )(acc, g, dinv, b.reshape(1, d_out))

    return out


# speedup vs baseline: 14.4800x; 1.2332x over previous
import functools

import jax
import jax.numpy as jnp
from jax import lax
from jax.experimental import pallas as pl
from jax.experimental.pallas import tpu as pltpu
from jax.experimental.pallas import tpu_sc as plsc

NC = 2
NS = 16
LANES = 16

CH = 128
UNROLL = 4


def _ceil_div(a, b):
    return (a + b - 1) // b


def _mesh():
    return plsc.VectorSubcoreMesh(core_axis_name="c", subcore_axis_name="s",
                                  num_cores=NC, num_subcores=NS)


def _zero_rows(buf, ref, row0, rows):
    full, rem = divmod(rows, CH)

    def chunk(j, _):
        pltpu.sync_copy(buf, ref.at[pl.ds(row0 + j * CH, CH)])
        return 0

    lax.fori_loop(0, full, chunk, 0)
    if rem:
        pltpu.sync_copy(buf.at[pl.ds(0, rem)],
                        ref.at[pl.ds(row0 + full * CH, rem)])


def _tile_range(total, workers, wid):
    lo = total // workers
    extra = total % workers
    start = wid * lo + jnp.minimum(wid, extra)
    count = lo + jnp.where(wid < extra, 1, 0)
    return start, count, lo + (1 if extra else 0)


def _make_deg_kernel(nchunks, half, deg_rows):
    lo = nchunks // NS
    max_ct = lo + (1 if nchunks % NS else 0)
    main = (lo // UNROLL) * UNROLL
    tile_rows = deg_rows // NS
    trash = half

    @functools.partial(
        pl.kernel,
        out_type=jax.ShapeDtypeStruct((NC * deg_rows, LANES), jnp.float32),
        mesh=_mesh(),
        compiler_params=pltpu.CompilerParams(use_tc_tiling_on_sc=False),
        scratch_types=[
            pltpu.VMEM((UNROLL, CH), jnp.int32),
            pltpu.VMEM((CH, LANES), jnp.float32),
            pltpu.VMEM((CH, LANES), jnp.float32),
            pltpu.VMEM_SHARED((deg_rows, LANES), jnp.float32),
        ] + [pltpu.SemaphoreType.DMA] * (2 * UNROLL),
    )
    def deg_kernel(dst_hbm, deg_out, idx_buf, ones_buf, zeros_buf, deg_sp,
                   *sems):
        isems = sems[:UNROLL]
        ssems = sems[UNROLL:]
        cid = lax.axis_index("c")
        sid = lax.axis_index("s")
        base = cid * half
        start, count, _ = _tile_range(nchunks, NS, sid)

        ones = jnp.full((LANES,), 1.0, dtype=jnp.float32)
        zeros = jnp.zeros((LANES,), dtype=jnp.float32)

        def init_row(r, _):
            ones_buf[r, :] = ones
            zeros_buf[r, :] = zeros
            return 0

        lax.fori_loop(0, CH, init_row, 0)
        _zero_rows(zeros_buf, deg_sp, sid * tile_rows, tile_rows)
        plsc.subcore_barrier()

        def remap_row(u):
            for q in range(CH // LANES):
                dloc = idx_buf[u, pl.ds(q * LANES, LANES)] - base
                ok = (dloc >= 0) & (dloc < half)
                idx_buf[u, pl.ds(q * LANES, LANES)] = jnp.where(ok, dloc,
                                                                trash)

        def step(t, _):
            c0 = (start + t * UNROLL) * CH
            ldescs = [
                pltpu.async_copy(dst_hbm.at[pl.ds(c0 + u * CH, CH)],
                                 idx_buf.at[u], isems[u])
                for u in range(UNROLL)
            ]
            sdescs = []
            for u in range(UNROLL):
                ldescs[u].wait()
                remap_row(u)
                sdescs.append(pltpu.async_copy(
                    ones_buf, deg_sp.at[idx_buf.at[u]], ssems[u], add=True))
            for sd in sdescs:
                sd.wait()
            return 0

        lax.fori_loop(0, main // UNROLL, step, 0)
        for j in range(main, max_ct):
            @pl.when(j < count)
            def _():
                pltpu.sync_copy(dst_hbm.at[pl.ds((start + j) * CH, CH)],
                                idx_buf.at[0])
                remap_row(0)
                pltpu.sync_copy(ones_buf, deg_sp.at[idx_buf.at[0]], add=True)
        plsc.subcore_barrier()

        pltpu.sync_copy(
            deg_sp.at[pl.ds(sid * tile_rows, tile_rows)],
            deg_out.at[pl.ds(cid * deg_rows + sid * tile_rows, tile_rows)])

    return deg_kernel


def _make_edge_kernel(nchunks, acc_rows, d):
    per_sc = nchunks // NC
    lo = per_sc // NS
    max_ct = lo + (1 if per_sc % NS else 0)
    main = (lo // UNROLL) * UNROLL
    tile_rows = acc_rows // NS

    @functools.partial(
        pl.kernel,
        out_type=jax.ShapeDtypeStruct((NC * acc_rows, d), jnp.bfloat16),
        mesh=_mesh(),
        compiler_params=pltpu.CompilerParams(use_tc_tiling_on_sc=False),
        scratch_types=[
            pltpu.VMEM((UNROLL, CH), jnp.int32),
            pltpu.VMEM((UNROLL, CH), jnp.int32),
            pltpu.VMEM((CH, d), jnp.bfloat16),
            pltpu.VMEM_SHARED((acc_rows, d), jnp.bfloat16),
        ] + [pltpu.VMEM((CH, d), jnp.bfloat16)] * UNROLL
          + [pltpu.SemaphoreType.DMA] * (3 * UNROLL),
    )
    def edge_kernel(src_hbm, dst_hbm, gbf_hbm, acc_out,
                    src_buf, dst_buf, zeros_buf, acc_sp, *rows_and_sems):
        rows = rows_and_sems[:UNROLL]
        isems = rows_and_sems[UNROLL:2 * UNROLL]
        gsems = rows_and_sems[2 * UNROLL:3 * UNROLL]
        ssems = rows_and_sems[3 * UNROLL:]
        cid = lax.axis_index("c")
        sid = lax.axis_index("s")
        start, count, _ = _tile_range(per_sc, NS, sid)
        start = cid * per_sc + start

        zeros = jnp.zeros((2 * LANES,), dtype=jnp.bfloat16)

        def init_row(r, _):
            for q in range(d // (2 * LANES)):
                zeros_buf[r, pl.ds(q * 2 * LANES, 2 * LANES)] = zeros
            return 0

        lax.fori_loop(0, CH, init_row, 0)
        _zero_rows(zeros_buf, acc_sp, sid * tile_rows, tile_rows)
        plsc.subcore_barrier()

        def step(t, _):
            c0 = (start + t * UNROLL) * CH
            ldescs = []
            for u in range(UNROLL):
                ldescs.append(pltpu.async_copy(
                    src_hbm.at[pl.ds(c0 + u * CH, CH)],
                    src_buf.at[u], isems[u]))
                ldescs.append(pltpu.async_copy(
                    dst_hbm.at[pl.ds(c0 + u * CH, CH)],
                    dst_buf.at[u], ssems[u]))
            gdescs = []
            for u in range(UNROLL):
                ldescs[2 * u].wait()
                gdescs.append(pltpu.async_copy(
                    gbf_hbm.at[src_buf.at[u]], rows[u], gsems[u]))
            sdescs = []
            for u in range(UNROLL):
                ldescs[2 * u + 1].wait()
                gdescs[u].wait()
                sdescs.append(pltpu.async_copy(
                    rows[u], acc_sp.at[dst_buf.at[u]], ssems[u], add=True))
            for sd in sdescs:
                sd.wait()
            return 0

        lax.fori_loop(0, main // UNROLL, step, 0)
        for j in range(main, max_ct):
            @pl.when(j < count)
            def _():
                pltpu.sync_copy(src_hbm.at[pl.ds((start + j) * CH, CH)],
                                src_buf.at[0])
                pltpu.sync_copy(dst_hbm.at[pl.ds((start + j) * CH, CH)],
                                dst_buf.at[0])
                pltpu.async_copy(gbf_hbm.at[src_buf.at[0]], rows[0],
                                 gsems[0]).wait()
                pltpu.sync_copy(rows[0], acc_sp.at[dst_buf.at[0]], add=True)
        plsc.subcore_barrier()

        pltpu.sync_copy(
            acc_sp.at[pl.ds(sid * tile_rows, tile_rows)],
            acc_out.at[pl.ds(cid * acc_rows + sid * tile_rows, tile_rows)])

    return edge_kernel


def _pre_body(x_ref, w_ref, deg_ref, g_ref, gbf_ref, dinv_ref):
    deg = deg_ref[...] + 1.0
    dinv = lax.rsqrt(deg)
    dinv_ref[...] = dinv
    g = jnp.dot(x_ref[...], w_ref[...],
                preferred_element_type=jnp.float32) * dinv
    g_ref[...] = g
    gbf_ref[...] = g.astype(jnp.bfloat16)


def _post_body(a0_ref, a1_ref, g_ref, dinv_ref, b_ref, o_ref):
    acc = (a0_ref[...].astype(jnp.float32)
           + a1_ref[...].astype(jnp.float32))
    o_ref[...] = jnp.tanh((acc + g_ref[...]) * dinv_ref[...] + b_ref[...])


BN = 128


def kernel(x, edge_index, W, b):
    n, d_in = x.shape
    d_out = W.shape[1]
    e = edge_index.shape[1]

    nb = _ceil_div(n, BN)
    npad = nb * BN

    assert e % CH == 0 and (e // CH) % NC == 0 and n % 2 == 0
    nchunks = e // CH
    half = n // 2
    deg_rows = _ceil_div(half + 1, NS) * NS
    acc_rows = _ceil_div(n, NS) * NS

    src = edge_index[0]
    dst = edge_index[1]

    deg_kernel = _make_deg_kernel(nchunks, half, deg_rows)
    deg_parts = deg_kernel(dst)
    degw = deg_parts.reshape(NC, deg_rows, LANES)
    deg = jnp.concatenate([degw[0, :half, 0:1], degw[1, :half, 0:1]], axis=0)

    g, gbf, dinv = pl.pallas_call(
        _pre_body,
        grid=(nb,),
        in_specs=[
            pl.BlockSpec((BN, d_in), lambda i: (i, 0)),
            pl.BlockSpec((d_in, d_out), lambda i: (0, 0)),
            pl.BlockSpec((BN, 1), lambda i: (i, 0)),
        ],
        out_specs=[
            pl.BlockSpec((BN, d_out), lambda i: (i, 0)),
            pl.BlockSpec((BN, d_out), lambda i: (i, 0)),
            pl.BlockSpec((BN, 1), lambda i: (i, 0)),
        ],
        out_shape=[
            jax.ShapeDtypeStruct((npad, d_out), jnp.float32),
            jax.ShapeDtypeStruct((npad, d_out), jnp.bfloat16),
            jax.ShapeDtypeStruct((npad, 1), jnp.float32),
        ],
    )(x, W, deg)

    edge_kernel = _make_edge_kernel(nchunks, acc_rows, d_out)
    accp = edge_kernel(src, dst, gbf).reshape(NC, acc_rows, d_out)

    out = pl.pallas_call(
        _post_body,
        grid=(nb,),
        in_specs=[
            pl.BlockSpec((BN, d_out), lambda i: (i, 0)),
            pl.BlockSpec((BN, d_out), lambda i: (i, 0)),
            pl.BlockSpec((BN, d_out), lambda i: (i, 0)),
            pl.BlockSpec((BN, 1), lambda i: (i, 0)),
            pl.BlockSpec((1, d_out), lambda i: (0, 0)),
        ],
        out_specs=pl.BlockSpec((BN, d_out), lambda i: (i, 0)),
        out_shape=jax.ShapeDtypeStruct((n, d_out), jnp.float32),
    )(accp[0], accp[1], g, dinv, b.reshape(1, d_out))

    return out


# speedup vs baseline: 14.6020x; 1.0084x over previous
import functools

import jax
import jax.numpy as jnp
from jax import lax
from jax.experimental import pallas as pl
from jax.experimental.pallas import tpu as pltpu
from jax.experimental.pallas import tpu_sc as plsc

NC = 2
NS = 16
LANES = 16

CH = 128
UNROLL = 6


def _ceil_div(a, b):
    return (a + b - 1) // b


def _mesh():
    return plsc.VectorSubcoreMesh(core_axis_name="c", subcore_axis_name="s",
                                  num_cores=NC, num_subcores=NS)


def _zero_rows(buf, ref, row0, rows):
    full, rem = divmod(rows, CH)

    def chunk(j, _):
        pltpu.sync_copy(buf, ref.at[pl.ds(row0 + j * CH, CH)])
        return 0

    lax.fori_loop(0, full, chunk, 0)
    if rem:
        pltpu.sync_copy(buf.at[pl.ds(0, rem)],
                        ref.at[pl.ds(row0 + full * CH, rem)])


def _tile_range(total, workers, wid):
    lo = total // workers
    extra = total % workers
    start = wid * lo + jnp.minimum(wid, extra)
    count = lo + jnp.where(wid < extra, 1, 0)
    return start, count, lo + (1 if extra else 0)


def _make_deg_kernel(nchunks, half, deg_rows):
    lo = nchunks // NS
    max_ct = lo + (1 if nchunks % NS else 0)
    main = (lo // UNROLL) * UNROLL
    tile_rows = deg_rows // NS
    trash = half

    @functools.partial(
        pl.kernel,
        out_type=jax.ShapeDtypeStruct((NC * deg_rows, LANES), jnp.float32),
        mesh=_mesh(),
        compiler_params=pltpu.CompilerParams(use_tc_tiling_on_sc=False),
        scratch_types=[
            pltpu.VMEM((UNROLL, CH), jnp.int32),
            pltpu.VMEM((CH, LANES), jnp.float32),
            pltpu.VMEM((CH, LANES), jnp.float32),
            pltpu.VMEM_SHARED((deg_rows, LANES), jnp.float32),
        ] + [pltpu.SemaphoreType.DMA] * (2 * UNROLL),
    )
    def deg_kernel(dst_hbm, deg_out, idx_buf, ones_buf, zeros_buf, deg_sp,
                   *sems):
        isems = sems[:UNROLL]
        ssems = sems[UNROLL:]
        cid = lax.axis_index("c")
        sid = lax.axis_index("s")
        base = cid * half
        start, count, _ = _tile_range(nchunks, NS, sid)

        ones = jnp.full((LANES,), 1.0, dtype=jnp.float32)
        zeros = jnp.zeros((LANES,), dtype=jnp.float32)

        def init_row(r, _):
            ones_buf[r, :] = ones
            zeros_buf[r, :] = zeros
            return 0

        lax.fori_loop(0, CH, init_row, 0)
        _zero_rows(zeros_buf, deg_sp, sid * tile_rows, tile_rows)
        plsc.subcore_barrier()

        def remap_row(u):
            for q in range(CH // LANES):
                dloc = idx_buf[u, pl.ds(q * LANES, LANES)] - base
                ok = (dloc >= 0) & (dloc < half)
                idx_buf[u, pl.ds(q * LANES, LANES)] = jnp.where(ok, dloc,
                                                                trash)

        def step(t, _):
            c0 = (start + t * UNROLL) * CH
            ldescs = [
                pltpu.async_copy(dst_hbm.at[pl.ds(c0 + u * CH, CH)],
                                 idx_buf.at[u], isems[u])
                for u in range(UNROLL)
            ]
            sdescs = []
            for u in range(UNROLL):
                ldescs[u].wait()
                remap_row(u)
                sdescs.append(pltpu.async_copy(
                    ones_buf, deg_sp.at[idx_buf.at[u]], ssems[u], add=True))
            for sd in sdescs:
                sd.wait()
            return 0

        lax.fori_loop(0, main // UNROLL, step, 0)
        for j in range(main, max_ct):
            @pl.when(j < count)
            def _():
                pltpu.sync_copy(dst_hbm.at[pl.ds((start + j) * CH, CH)],
                                idx_buf.at[0])
                remap_row(0)
                pltpu.sync_copy(ones_buf, deg_sp.at[idx_buf.at[0]], add=True)
        plsc.subcore_barrier()

        pltpu.sync_copy(
            deg_sp.at[pl.ds(sid * tile_rows, tile_rows)],
            deg_out.at[pl.ds(cid * deg_rows + sid * tile_rows, tile_rows)])

    return deg_kernel


def _make_edge_kernel(nchunks, acc_rows, d):
    per_sc = nchunks // NC
    lo = per_sc // NS
    max_ct = lo + (1 if per_sc % NS else 0)
    main = (lo // UNROLL) * UNROLL
    tile_rows = acc_rows // NS

    @functools.partial(
        pl.kernel,
        out_type=jax.ShapeDtypeStruct((NC * acc_rows, d), jnp.bfloat16),
        mesh=_mesh(),
        compiler_params=pltpu.CompilerParams(use_tc_tiling_on_sc=False),
        scratch_types=[
            pltpu.VMEM((UNROLL, CH), jnp.int32),
            pltpu.VMEM((UNROLL, CH), jnp.int32),
            pltpu.VMEM((CH, d), jnp.bfloat16),
            pltpu.VMEM_SHARED((acc_rows, d), jnp.bfloat16),
        ] + [pltpu.VMEM((CH, d), jnp.bfloat16)] * UNROLL
          + [pltpu.SemaphoreType.DMA] * (3 * UNROLL),
    )
    def edge_kernel(src_hbm, dst_hbm, gbf_hbm, acc_out,
                    src_buf, dst_buf, zeros_buf, acc_sp, *rows_and_sems):
        rows = rows_and_sems[:UNROLL]
        isems = rows_and_sems[UNROLL:2 * UNROLL]
        gsems = rows_and_sems[2 * UNROLL:3 * UNROLL]
        ssems = rows_and_sems[3 * UNROLL:]
        cid = lax.axis_index("c")
        sid = lax.axis_index("s")
        start, count, _ = _tile_range(per_sc, NS, sid)
        start = cid * per_sc + start

        zeros = jnp.zeros((2 * LANES,), dtype=jnp.bfloat16)

        def init_row(r, _):
            for q in range(d // (2 * LANES)):
                zeros_buf[r, pl.ds(q * 2 * LANES, 2 * LANES)] = zeros
            return 0

        lax.fori_loop(0, CH, init_row, 0)
        _zero_rows(zeros_buf, acc_sp, sid * tile_rows, tile_rows)
        plsc.subcore_barrier()

        def step(t, _):
            c0 = (start + t * UNROLL) * CH
            ldescs = []
            for u in range(UNROLL):
                ldescs.append(pltpu.async_copy(
                    src_hbm.at[pl.ds(c0 + u * CH, CH)],
                    src_buf.at[u], isems[u]))
                ldescs.append(pltpu.async_copy(
                    dst_hbm.at[pl.ds(c0 + u * CH, CH)],
                    dst_buf.at[u], ssems[u]))
            gdescs = []
            for u in range(UNROLL):
                ldescs[2 * u].wait()
                gdescs.append(pltpu.async_copy(
                    gbf_hbm.at[src_buf.at[u]], rows[u], gsems[u]))
            sdescs = []
            for u in range(UNROLL):
                ldescs[2 * u + 1].wait()
                gdescs[u].wait()
                sdescs.append(pltpu.async_copy(
                    rows[u], acc_sp.at[dst_buf.at[u]], ssems[u], add=True))
            for sd in sdescs:
                sd.wait()
            return 0

        lax.fori_loop(0, main // UNROLL, step, 0)
        for j in range(main, max_ct):
            @pl.when(j < count)
            def _():
                pltpu.sync_copy(src_hbm.at[pl.ds((start + j) * CH, CH)],
                                src_buf.at[0])
                pltpu.sync_copy(dst_hbm.at[pl.ds((start + j) * CH, CH)],
                                dst_buf.at[0])
                pltpu.async_copy(gbf_hbm.at[src_buf.at[0]], rows[0],
                                 gsems[0]).wait()
                pltpu.sync_copy(rows[0], acc_sp.at[dst_buf.at[0]], add=True)
        plsc.subcore_barrier()

        pltpu.sync_copy(
            acc_sp.at[pl.ds(sid * tile_rows, tile_rows)],
            acc_out.at[pl.ds(cid * acc_rows + sid * tile_rows, tile_rows)])

    return edge_kernel


def _pre_body(x_ref, w_ref, deg_ref, g_ref, gbf_ref, dinv_ref):
    deg = deg_ref[...] + 1.0
    dinv = lax.rsqrt(deg)
    dinv_ref[...] = dinv
    g = jnp.dot(x_ref[...], w_ref[...],
                preferred_element_type=jnp.float32) * dinv
    g_ref[...] = g
    gbf_ref[...] = g.astype(jnp.bfloat16)


def _post_body(a0_ref, a1_ref, g_ref, dinv_ref, b_ref, o_ref):
    acc = (a0_ref[...].astype(jnp.float32)
           + a1_ref[...].astype(jnp.float32))
    o_ref[...] = jnp.tanh((acc + g_ref[...]) * dinv_ref[...] + b_ref[...])


BN = 128


def kernel(x, edge_index, W, b):
    n, d_in = x.shape
    d_out = W.shape[1]
    e = edge_index.shape[1]

    nb = _ceil_div(n, BN)
    npad = nb * BN

    assert e % CH == 0 and (e // CH) % NC == 0 and n % 2 == 0
    nchunks = e // CH
    half = n // 2
    deg_rows = _ceil_div(half + 1, NS) * NS
    acc_rows = _ceil_div(n, NS) * NS

    src = edge_index[0]
    dst = edge_index[1]

    deg_kernel = _make_deg_kernel(nchunks, half, deg_rows)
    deg_parts = deg_kernel(dst)
    degw = deg_parts.reshape(NC, deg_rows, LANES)
    deg = jnp.concatenate([degw[0, :half, 0:1], degw[1, :half, 0:1]], axis=0)

    g, gbf, dinv = pl.pallas_call(
        _pre_body,
        grid=(nb,),
        in_specs=[
            pl.BlockSpec((BN, d_in), lambda i: (i, 0)),
            pl.BlockSpec((d_in, d_out), lambda i: (0, 0)),
            pl.BlockSpec((BN, 1), lambda i: (i, 0)),
        ],
        out_specs=[
            pl.BlockSpec((BN, d_out), lambda i: (i, 0)),
            pl.BlockSpec((BN, d_out), lambda i: (i, 0)),
            pl.BlockSpec((BN, 1), lambda i: (i, 0)),
        ],
        out_shape=[
            jax.ShapeDtypeStruct((npad, d_out), jnp.float32),
            jax.ShapeDtypeStruct((npad, d_out), jnp.bfloat16),
            jax.ShapeDtypeStruct((npad, 1), jnp.float32),
        ],
    )(x, W, deg)

    edge_kernel = _make_edge_kernel(nchunks, acc_rows, d_out)
    accp = edge_kernel(src, dst, gbf).reshape(NC, acc_rows, d_out)

    out = pl.pallas_call(
        _post_body,
        grid=(nb,),
        in_specs=[
            pl.BlockSpec((BN, d_out), lambda i: (i, 0)),
            pl.BlockSpec((BN, d_out), lambda i: (i, 0)),
            pl.BlockSpec((BN, d_out), lambda i: (i, 0)),
            pl.BlockSpec((BN, 1), lambda i: (i, 0)),
            pl.BlockSpec((1, d_out), lambda i: (0, 0)),
        ],
        out_specs=pl.BlockSpec((BN, d_out), lambda i: (i, 0)),
        out_shape=jax.ShapeDtypeStruct((n, d_out), jnp.float32),
    )(accp[0], accp[1], g, dinv, b.reshape(1, d_out))

    return out
